# baseline (device time: 45974 ns/iter reference)
import jax
import jax.numpy as jnp
from jax import lax
from jax.experimental import pallas as pl
from jax.experimental.pallas import tpu as pltpu

Z = 4
BLK = 512


def kernel(x, dy, gamma):
    m, d = x.shape
    grid = m // BLK

    def body(x_ref, dy_ref, out_ref, acc_ref, comm_ref, vmem_filler_ref,
             send_sems, recv_sems):
        step = pl.program_id(0)
        my_x = lax.axis_index("x")
        my_y = lax.axis_index("y")
        my_z = lax.axis_index("z")
        barrier_sem = pltpu.get_barrier_semaphore()

        @pl.when(step == 0)
        def _():
            vmem_filler_ref[0:1, :] = jnp.zeros((1, 128), jnp.float32)
            for off in range(1, Z):
                pl.semaphore_signal(
                    barrier_sem,
                    inc=1,
                    device_id=(my_x, my_y, lax.rem(my_z + off, Z)),
                    device_id_type=pl.DeviceIdType.MESH,
                )

        xv = x_ref[:, :]
        dyv = dy_ref[:, :]
        ones_d = jnp.ones((d, 1), jnp.float32)
        dn_row = (((1,), (0,)), ((), ()))
        s1 = lax.dot_general(xv, ones_d, dn_row, precision="highest")
        s2 = lax.dot_general(xv * xv, ones_d, dn_row, precision="highest")
        mu = s1 * (1.0 / d)
        var = s2 * (1.0 / d) - mu * mu
        rstd = lax.rsqrt(var + 1e-5)
        b = rstd * mu
        dgamma = jnp.sum(dyv * (rstd * xv - b), axis=0)[None, :]
        dbeta = jnp.sum(dyv, axis=0)[None, :]
        part = jnp.concatenate([dgamma, dbeta], axis=0)

        @pl.when(step == 0)
        def _():
            acc_ref[:, :] = part

        @pl.when(step != 0)
        def _():
            acc_ref[:, :] = acc_ref[:, :] + part

        @pl.when(step == grid - 1)
        def _():
            comm_ref[pl.ds(my_z, 1)] = acc_ref[:, :][None]
            pl.semaphore_wait(barrier_sem, Z - 1)

            sends = []
            for off in range(1, Z):
                rdma = pltpu.make_async_remote_copy(
                    src_ref=comm_ref.at[my_z],
                    dst_ref=comm_ref.at[my_z],
                    send_sem=send_sems.at[off - 1],
                    recv_sem=recv_sems.at[my_z],
                    device_id=(my_x, my_y, lax.rem(my_z + off, Z)),
                    device_id_type=pl.DeviceIdType.MESH,
                )
                rdma.start()
                sends.append(rdma)

            for off in range(1, Z):
                src_z = lax.rem(my_z + Z - off, Z)
                recv = pltpu.make_async_remote_copy(
                    src_ref=comm_ref.at[src_z],
                    dst_ref=comm_ref.at[src_z],
                    send_sem=send_sems.at[off - 1],
                    recv_sem=recv_sems.at[src_z],
                    device_id=(my_x, my_y, my_z),
                    device_id_type=pl.DeviceIdType.MESH,
                )
                recv.wait_recv()
            for s in sends:
                s.wait_send()

            out_ref[:, :] = (
                comm_ref[0] + comm_ref[1] + comm_ref[2] + comm_ref[3]
            )

    return pl.pallas_call(
        body,
        grid=(grid,),
        out_shape=jax.ShapeDtypeStruct((2, d), jnp.float32),
        in_specs=[
            pl.BlockSpec((BLK, d), lambda i: (i, 0)),
            pl.BlockSpec((BLK, d), lambda i: (i, 0)),
        ],
        out_specs=pl.BlockSpec((2, d), lambda i: (0, 0)),
        scratch_shapes=[
            pltpu.VMEM((2, d), jnp.float32),
            pltpu.VMEM((Z, 2, d), jnp.float32),
            pltpu.VMEM((80 * 1024, 128), jnp.float32),
            pltpu.SemaphoreType.DMA((Z - 1,)),
            pltpu.SemaphoreType.DMA((Z,)),
        ],
        compiler_params=pltpu.CompilerParams(collective_id=0),
    )(x, dy)


# device time: 14263 ns/iter; 3.2233x vs baseline; 3.2233x over previous
import jax
import jax.numpy as jnp
from jax import lax
from jax.experimental import pallas as pl
from jax.experimental.pallas import tpu as pltpu

NX, NY, NZ = 2, 2, 4
NDEV = NX * NY * NZ
QUARTERS = NX * NY
BLK = 256


def kernel(x, dy, gamma):
    m, d = x.shape
    qrows = m // QUARTERS
    grid = qrows // BLK

    def body(x_hbm, dy_hbm, out_hbm, xbuf, dybuf, acc_ref, comm_ref,
             sum_ref, in_sems, out_sem, send_sems, recv_sems):
        step = pl.program_id(0)
        slot = lax.rem(step, 2)
        my_x = lax.axis_index("x")
        my_y = lax.axis_index("y")
        my_z = lax.axis_index("z")
        q = my_x * 2 + my_y
        my_id = q * NZ + my_z
        row0 = q * qrows
        barrier_sem = pltpu.get_barrier_semaphore()

        def others():
            for dx in range(NX):
                for dyy in range(NY):
                    for dz in range(NZ):
                        if dx == 0 and dyy == 0 and dz == 0:
                            continue
                        yield (
                            lax.rem(my_x + dx, NX),
                            lax.rem(my_y + dyy, NY),
                            lax.rem(my_z + dz, NZ),
                        )

        def in_copies(blk_idx, buf_slot):
            rows = pl.ds(row0 + blk_idx * BLK, BLK)
            return (
                pltpu.make_async_copy(
                    x_hbm.at[rows, :], xbuf.at[buf_slot], in_sems.at[0, buf_slot]),
                pltpu.make_async_copy(
                    dy_hbm.at[rows, :], dybuf.at[buf_slot], in_sems.at[1, buf_slot]),
            )

        @pl.when(step == 0)
        def _():
            for c in in_copies(0, 0):
                c.start()
            for tgt in others():
                pl.semaphore_signal(
                    barrier_sem, inc=1, device_id=tgt,
                    device_id_type=pl.DeviceIdType.MESH,
                )

        @pl.when(step < grid - 1)
        def _():
            for c in in_copies(step + 1, 1 - slot):
                c.start()

        for c in in_copies(step, slot):
            c.wait()

        xv = xbuf[slot]
        dyv = dybuf[slot]
        s1 = jnp.sum(xv, axis=1, keepdims=True)
        s2 = jnp.sum(xv * xv, axis=1, keepdims=True)
        mu = s1 * (1.0 / d)
        var = s2 * (1.0 / d) - mu * mu
        rstd = lax.rsqrt(var + 1e-5)
        b = rstd * mu
        dgamma = jnp.sum(dyv * (rstd * xv - b), axis=0)[None, :]
        dbeta = jnp.sum(dyv, axis=0)[None, :]
        part = jnp.concatenate([dgamma, dbeta], axis=0)

        @pl.when(step == 0)
        def _():
            acc_ref[:, :] = part

        @pl.when(step != 0)
        def _():
            acc_ref[:, :] = acc_ref[:, :] + part

        @pl.when(step == grid - 1)
        def _():
            comm_ref[pl.ds(my_id, 1)] = acc_ref[:, :][None]
            pl.semaphore_wait(barrier_sem, NDEV - 1)

            sends = []
            for i, tgt in enumerate(others()):
                rdma = pltpu.make_async_remote_copy(
                    src_ref=comm_ref.at[my_id],
                    dst_ref=comm_ref.at[my_id],
                    send_sem=send_sems.at[i],
                    recv_sem=recv_sems.at[my_id],
                    device_id=tgt,
                    device_id_type=pl.DeviceIdType.MESH,
                )
                rdma.start()
                sends.append(rdma)

            for src in range(1, NDEV):
                src_id = lax.rem(my_id + src, NDEV)
                recv = pltpu.make_async_remote_copy(
                    src_ref=comm_ref.at[src_id],
                    dst_ref=comm_ref.at[src_id],
                    send_sem=send_sems.at[src - 1],
                    recv_sem=recv_sems.at[src_id],
                    device_id=(my_x, my_y, my_z),
                    device_id_type=pl.DeviceIdType.MESH,
                )
                recv.wait_recv()
            for s in sends:
                s.wait_send()

            total = comm_ref[0]
            for s in range(1, NDEV):
                total = total + comm_ref[s]
            sum_ref[:, :] = total
            out_copy = pltpu.make_async_copy(sum_ref, out_hbm, out_sem)
            out_copy.start()
            out_copy.wait()

    return pl.pallas_call(
        body,
        grid=(grid,),
        out_shape=jax.ShapeDtypeStruct((2, d), jnp.float32),
        in_specs=[
            pl.BlockSpec(memory_space=pltpu.MemorySpace.HBM),
            pl.BlockSpec(memory_space=pltpu.MemorySpace.HBM),
        ],
        out_specs=pl.BlockSpec(memory_space=pltpu.MemorySpace.HBM),
        scratch_shapes=[
            pltpu.VMEM((2, BLK, d), jnp.float32),
            pltpu.VMEM((2, BLK, d), jnp.float32),
            pltpu.VMEM((2, d), jnp.float32),
            pltpu.VMEM((NDEV, 2, d), jnp.float32),
            pltpu.VMEM((2, d), jnp.float32),
            pltpu.SemaphoreType.DMA((2, 2)),
            pltpu.SemaphoreType.DMA,
            pltpu.SemaphoreType.DMA((NDEV - 1,)),
            pltpu.SemaphoreType.DMA((NDEV,)),
        ],
        compiler_params=pltpu.CompilerParams(collective_id=0),
    )(
        pltpu.with_memory_space_constraint(x, pltpu.MemorySpace.HBM),
        pltpu.with_memory_space_constraint(dy, pltpu.MemorySpace.HBM),
    )


# device time: 13973 ns/iter; 3.2902x vs baseline; 1.0208x over previous
import jax
import jax.numpy as jnp
from jax import lax
from jax.experimental import pallas as pl
from jax.experimental.pallas import tpu as pltpu

NX, NY, NZ = 2, 2, 4
NDEV = NX * NY * NZ
QUARTERS = NX * NY
BLK = 256


def kernel(x, dy, gamma):
    m, d = x.shape
    qrows = m // QUARTERS
    grid = qrows // BLK

    def body(x_hbm, dy_hbm, out_hbm, xbuf, dybuf, acc_ref, commz_ref,
             commq_ref, sum_ref, in_sems, out_sem, send_z_sems,
             recv_z_sems, send_q_sems, recv_q_sems):
        step = pl.program_id(0)
        slot = lax.rem(step, 2)
        my_x = lax.axis_index("x")
        my_y = lax.axis_index("y")
        my_z = lax.axis_index("z")
        q = my_x * 2 + my_y
        my_id = q * NZ + my_z
        row0 = q * qrows
        barrier_sem = pltpu.get_barrier_semaphore()

        z_peers = [
            (my_x, my_y, lax.rem(my_z + dz, NZ)) for dz in range(1, NZ)
        ]
        q_peers = [
            (lax.rem(my_x + dx, NX), lax.rem(my_y + dyy, NY), my_z)
            for dx in range(NX) for dyy in range(NY)
            if (dx, dyy) != (0, 0)
        ]

        def in_copies(blk_idx, buf_slot):
            rows = pl.ds(row0 + blk_idx * BLK, BLK)
            return (
                pltpu.make_async_copy(
                    x_hbm.at[rows, :], xbuf.at[buf_slot], in_sems.at[0, buf_slot]),
                pltpu.make_async_copy(
                    dy_hbm.at[rows, :], dybuf.at[buf_slot], in_sems.at[1, buf_slot]),
            )

        @pl.when(step == 0)
        def _():
            for c in in_copies(0, 0):
                c.start()
            for tgt in z_peers + q_peers:
                pl.semaphore_signal(
                    barrier_sem, inc=1, device_id=tgt,
                    device_id_type=pl.DeviceIdType.MESH,
                )

        @pl.when(step < grid - 1)
        def _():
            for c in in_copies(step + 1, 1 - slot):
                c.start()

        for c in in_copies(step, slot):
            c.wait()

        xv = xbuf[slot]
        dyv = dybuf[slot]
        s1 = jnp.sum(xv, axis=1, keepdims=True)
        s2 = jnp.sum(xv * xv, axis=1, keepdims=True)
        mu = s1 * (1.0 / d)
        var = s2 * (1.0 / d) - mu * mu
        rstd = lax.rsqrt(var + 1e-5)
        b = rstd * mu
        dgamma = jnp.sum(dyv * (rstd * xv - b), axis=0)[None, :]
        dbeta = jnp.sum(dyv, axis=0)[None, :]
        part = jnp.concatenate([dgamma, dbeta], axis=0)

        @pl.when(step == 0)
        def _():
            acc_ref[:, :] = part

        @pl.when(step != 0)
        def _():
            acc_ref[:, :] = acc_ref[:, :] + part

        @pl.when(step == grid - 1)
        def _():
            commz_ref[pl.ds(my_z, 1)] = acc_ref[:, :][None]
            pl.semaphore_wait(barrier_sem, len(z_peers) + len(q_peers))

            def exchange(buf_ref, my_slot, peers, n_slots, send_sems,
                         recv_sems):
                sends = []
                for i, tgt in enumerate(peers):
                    rdma = pltpu.make_async_remote_copy(
                        src_ref=buf_ref.at[my_slot],
                        dst_ref=buf_ref.at[my_slot],
                        send_sem=send_sems.at[i],
                        recv_sem=recv_sems.at[my_slot],
                        device_id=tgt,
                        device_id_type=pl.DeviceIdType.MESH,
                    )
                    rdma.start()
                    sends.append(rdma)
                for src in range(1, n_slots):
                    src_slot = lax.rem(my_slot + src, n_slots)
                    recv = pltpu.make_async_remote_copy(
                        src_ref=buf_ref.at[src_slot],
                        dst_ref=buf_ref.at[src_slot],
                        send_sem=send_sems.at[src - 1],
                        recv_sem=recv_sems.at[src_slot],
                        device_id=(my_x, my_y, my_z),
                        device_id_type=pl.DeviceIdType.MESH,
                    )
                    recv.wait_recv()
                total = buf_ref[0]
                for s in range(1, n_slots):
                    total = total + buf_ref[s]
                for s in sends:
                    s.wait_send()
                return total

            zsum = exchange(commz_ref, my_z, z_peers, NZ,
                            send_z_sems, recv_z_sems)
            commq_ref[pl.ds(q, 1)] = zsum[None]
            total = exchange(commq_ref, q, q_peers, QUARTERS,
                             send_q_sems, recv_q_sems)

            sum_ref[:, :] = total
            out_copy = pltpu.make_async_copy(sum_ref, out_hbm, out_sem)
            out_copy.start()
            out_copy.wait()

    return pl.pallas_call(
        body,
        grid=(grid,),
        out_shape=jax.ShapeDtypeStruct((2, d), jnp.float32),
        in_specs=[
            pl.BlockSpec(memory_space=pltpu.MemorySpace.HBM),
            pl.BlockSpec(memory_space=pltpu.MemorySpace.HBM),
        ],
        out_specs=pl.BlockSpec(memory_space=pltpu.MemorySpace.HBM),
        scratch_shapes=[
            pltpu.VMEM((2, BLK, d), jnp.float32),
            pltpu.VMEM((2, BLK, d), jnp.float32),
            pltpu.VMEM((2, d), jnp.float32),
            pltpu.VMEM((NZ, 2, d), jnp.float32),
            pltpu.VMEM((QUARTERS, 2, d), jnp.float32),
            pltpu.VMEM((2, d), jnp.float32),
            pltpu.SemaphoreType.DMA((2, 2)),
            pltpu.SemaphoreType.DMA,
            pltpu.SemaphoreType.DMA((NZ - 1,)),
            pltpu.SemaphoreType.DMA((NZ,)),
            pltpu.SemaphoreType.DMA((QUARTERS - 1,)),
            pltpu.SemaphoreType.DMA((QUARTERS,)),
        ],
        compiler_params=pltpu.CompilerParams(collective_id=0),
    )(
        pltpu.with_memory_space_constraint(x, pltpu.MemorySpace.HBM),
        pltpu.with_memory_space_constraint(dy, pltpu.MemorySpace.HBM),
    )
